# baseline (speedup 1.0000x reference)
import functools

import jax
import jax.numpy as jnp
from jax import lax
from jax.experimental import pallas as pl
from jax.experimental.pallas import tpu as pltpu
from jax.experimental.pallas import tpu_sc as plsc

NUM_SAMPLES = 32
C_FRAMES = 3
T = 300
H = 224
W = 224
NC = 2
NFRAMES = C_FRAMES * NUM_SAMPLES
K = NFRAMES // NC
NBUF = 8
D = 4


def _sc_subsample(x):
    mesh = plsc.ScalarSubcoreMesh(axis_name="c", num_cores=NC)

    @functools.partial(
        pl.kernel,
        mesh=mesh,
        out_type=jax.ShapeDtypeStruct((C_FRAMES, NUM_SAMPLES, H, W), jnp.float32),
        scratch_types=[pltpu.VMEM_SHARED((NBUF, 1, 1, H, W), jnp.float32),
                       pltpu.SemaphoreType.DMA,
                       pltpu.SemaphoreType.DMA],
    )
    def k(x_hbm, out_hbm, bufs, rsem, wsem):
        w = lax.axis_index("c")

        def frame_loc(i):
            f = w * K + i
            j = f % NUM_SAMPLES
            c = f // NUM_SAMPLES
            t = (299 * j) // 31
            return c, j, t

        reads = [None] * K
        writes = [None] * K
        for i in range(K + D):
            if i < K:
                b = i % NBUF
                if i >= NBUF:
                    writes[i - NBUF].wait()
                c, j, t = frame_loc(i)
                reads[i] = pltpu.async_copy(
                    x_hbm.at[pl.ds(c, 1), pl.ds(t, 1)], bufs.at[b], rsem
                )
            wi = i - D
            if wi >= 0:
                reads[wi].wait()
                pc, pj, _ = frame_loc(wi)
                writes[wi] = pltpu.async_copy(
                    bufs.at[wi % NBUF],
                    out_hbm.at[pl.ds(pc, 1), pl.ds(pj, 1)],
                    wsem,
                )
        for i in range(max(0, K - NBUF), K):
            writes[i].wait()

    return k(x)


def kernel(x):
    return _sc_subsample(x)
